# baseline (device time: 571579 ns/iter reference)
import jax
import jax.numpy as jnp
from jax import lax
from jax.experimental import pallas as pl
from jax.experimental.pallas import tpu as pltpu

N_DEV = 32


def _gelu(y):
    c = 0.7978845608028654
    return 0.5 * y * (1.0 + jnp.tanh(c * (y + 0.044715 * y * y * y)))


def kernel(x, w_mat):
    m_total, k_per = x.shape
    k_per2, n = w_mat.shape
    assert k_per == k_per2
    m_per = m_total // N_DEV

    def body(x_ref, w_ref, out_ref, buf, send_sems, recv_sems, credit_sem):
        my = lax.axis_index("i")
        left = jnp.mod(my - 1, N_DEV)
        right = jnp.mod(my + 1, N_DEV)

        barrier_sem = pltpu.get_barrier_semaphore()
        for nbr in (left, right):
            pl.semaphore_signal(
                barrier_sem, inc=1,
                device_id=(nbr,), device_id_type=pl.DeviceIdType.MESH,
            )
        pl.semaphore_wait(barrier_sem, 2)

        def partial_chunk(c):
            xs = x_ref[pl.ds(c * m_per, m_per), :]
            return jnp.dot(xs, w_ref[:, :], preferred_element_type=jnp.float32)

        buf[0, :, :] = partial_chunk(jnp.mod(my - 1, N_DEV))

        for s in range(N_DEV - 1):
            send_slot = s % 2
            recv_slot = (s + 1) % 2
            if s >= 1:
                pl.semaphore_wait(credit_sem, 1)
            rdma = pltpu.make_async_remote_copy(
                src_ref=buf.at[send_slot],
                dst_ref=buf.at[recv_slot],
                send_sem=send_sems.at[send_slot],
                recv_sem=recv_sems.at[recv_slot],
                device_id=(right,),
                device_id_type=pl.DeviceIdType.MESH,
            )
            rdma.start()
            rdma.wait()
            if s < N_DEV - 2:
                pl.semaphore_signal(
                    credit_sem, inc=1,
                    device_id=(left,), device_id_type=pl.DeviceIdType.MESH,
                )
            c = jnp.mod(my - 2 - s, N_DEV)
            if s < N_DEV - 2:
                buf[recv_slot, :, :] = buf[recv_slot, :, :] + partial_chunk(c)
            else:
                y = buf[recv_slot, :, :] + partial_chunk(c)
                out_ref[:, :] = _gelu(y)

    return pl.pallas_call(
        body,
        out_shape=jax.ShapeDtypeStruct((m_per, n), jnp.float32),
        in_specs=[
            pl.BlockSpec(memory_space=pltpu.VMEM),
            pl.BlockSpec(memory_space=pltpu.VMEM),
        ],
        out_specs=pl.BlockSpec(memory_space=pltpu.VMEM),
        scratch_shapes=[
            pltpu.VMEM((2, m_per, n), jnp.float32),
            pltpu.SemaphoreType.DMA((2,)),
            pltpu.SemaphoreType.DMA((2,)),
            pltpu.SemaphoreType.REGULAR,
        ],
        compiler_params=pltpu.CompilerParams(collective_id=0),
    )(x, w_mat)


# device time: 360568 ns/iter; 1.5852x vs baseline; 1.5852x over previous
import jax
import jax.numpy as jnp
from jax import lax
from jax.experimental import pallas as pl
from jax.experimental.pallas import tpu as pltpu

N_DEV = 32
NB = 2
NBLK = 2 * NB


def _gelu(y):
    c = 0.7978845608028654
    return 0.5 * y * (1.0 + jnp.tanh(c * (y + 0.044715 * y * y * y)))


def kernel(x, w_mat):
    m_total, k_per = x.shape
    _, n = w_mat.shape
    m_per = m_total // N_DEV
    width = n // NBLK

    def body(x_ref, w_ref, out_ref, buf, send_sems, recv_sems,
             credit_r, credit_l):
        my = lax.axis_index("i")
        left = jnp.mod(my - 1, N_DEV)
        right = jnp.mod(my + 1, N_DEV)

        barrier_sem = pltpu.get_barrier_semaphore()
        for nbr in (left, right):
            pl.semaphore_signal(
                barrier_sem, inc=1,
                device_id=(nbr,), device_id_type=pl.DeviceIdType.MESH,
            )
        pl.semaphore_wait(barrier_sem, 2)

        def partial(c, idx):
            xs = x_ref[pl.ds(c * m_per, m_per), :]
            ws = w_ref[:, pl.ds(idx * width, width)]
            return jnp.dot(xs, ws, preferred_element_type=jnp.float32)

        def send_chunk(idx, s):
            if idx < NB:
                return jnp.mod(my - 1 - s, N_DEV)
            return jnp.mod(my + 1 + s, N_DEV)

        def recv_chunk(idx, s):
            if idx < NB:
                return jnp.mod(my - 2 - s, N_DEV)
            return jnp.mod(my + 2 + s, N_DEV)

        def desc(idx, s):
            to = right if idx < NB else left
            return pltpu.make_async_remote_copy(
                src_ref=buf.at[s % 2, idx],
                dst_ref=buf.at[(s + 1) % 2, idx],
                send_sem=send_sems.at[s % 2, idx],
                recv_sem=recv_sems.at[(s + 1) % 2, idx],
                device_id=(to,),
                device_id_type=pl.DeviceIdType.MESH,
            )

        def credit_sem(idx):
            return credit_r if idx < NB else credit_l

        def upstream(idx):
            return left if idx < NB else right

        for idx in range(NBLK):
            buf[0, idx] = partial(send_chunk(idx, 0), idx)

        for s in range(N_DEV - 1):
            for idx in range(NBLK):
                if s >= 1:
                    desc(idx, s - 1).wait()
                    pl.semaphore_signal(
                        credit_sem(idx), inc=1,
                        device_id=(upstream(idx),),
                        device_id_type=pl.DeviceIdType.MESH,
                    )
                    buf[s % 2, idx] = buf[s % 2, idx] + partial(
                        recv_chunk(idx, s - 1), idx)
                    pl.semaphore_wait(credit_sem(idx), 1)
                desc(idx, s).start()

        for idx in range(NBLK):
            desc(idx, N_DEV - 2).wait()
            y = buf[(N_DEV - 1) % 2, idx] + partial(my, idx)
            out_ref[:, pl.ds(idx * width, width)] = _gelu(y)

    return pl.pallas_call(
        body,
        out_shape=jax.ShapeDtypeStruct((m_per, n), jnp.float32),
        in_specs=[
            pl.BlockSpec(memory_space=pltpu.VMEM),
            pl.BlockSpec(memory_space=pltpu.VMEM),
        ],
        out_specs=pl.BlockSpec(memory_space=pltpu.VMEM),
        scratch_shapes=[
            pltpu.VMEM((2, NBLK, m_per, width), jnp.float32),
            pltpu.SemaphoreType.DMA((2, NBLK)),
            pltpu.SemaphoreType.DMA((2, NBLK)),
            pltpu.SemaphoreType.REGULAR,
            pltpu.SemaphoreType.REGULAR,
        ],
        compiler_params=pltpu.CompilerParams(collective_id=0),
    )(x, w_mat)
